# baseline (speedup 1.0000x reference)
import functools

import jax
import jax.numpy as jnp
from jax.experimental import pallas as pl
from jax.experimental.pallas import tpu as pltpu

NBUF = 4
LOOKAHEAD = 3


def _body(P1, P2, R1, R2, e_ref, qc_ref, cq_ref, out_ref,
          pool1, pool2, e_bf, msg_ref, sem1, sem2):
    i = pl.program_id(0)

    def qc_copy(b):
        slot = jax.lax.rem(b, NBUF)
        return pltpu.make_async_copy(
            qc_ref.at[pl.ds(b * R1, R1), :], pool1.at[slot], sem1.at[slot]
        )

    def cq_copy(b):
        slot = jax.lax.rem(b, NBUF)
        return pltpu.make_async_copy(
            cq_ref.at[pl.ds(b * R2, R2), :], pool2.at[slot], sem2.at[slot]
        )

    @pl.when(i == 0)
    def _warmup():
        for b in range(min(LOOKAHEAD, P1)):
            qc_copy(b).start()
        e_bf[...] = e_ref[...].astype(jnp.bfloat16)

    @pl.when(i < P1)
    def _p1():
        @pl.when(i + LOOKAHEAD < P1)
        def _():
            qc_copy(i + LOOKAHEAD).start()

        @pl.when(i >= P1 - min(LOOKAHEAD, P2))
        def _():
            cq_copy(i - (P1 - min(LOOKAHEAD, P2))).start()

        qc_copy(i).wait()
        acc = jnp.dot(
            pool1[jax.lax.rem(i, NBUF)].astype(jnp.bfloat16),
            e_bf[...],
            preferred_element_type=jnp.float32,
        )
        msg_ref[pl.ds(i * R1, R1), :] = acc.astype(jnp.bfloat16)

    @pl.when(i >= P1)
    def _p2():
        j = i - P1

        @pl.when(j + LOOKAHEAD < P2)
        def _():
            cq_copy(j + LOOKAHEAD).start()

        cq_copy(j).wait()
        out_ref[pl.ds(j * R2, R2), :] = jnp.dot(
            pool2[jax.lax.rem(j, NBUF)].astype(jnp.bfloat16),
            msg_ref[...],
            preferred_element_type=jnp.float32,
        )


@jax.jit
def kernel(skill_embs, HG_qc, HG_cq):
    n_edges, n_skills = HG_qc.shape
    d = skill_embs.shape[1]
    R1 = 64
    R2 = 256
    P1 = n_edges // R1
    P2 = n_skills // R2

    return pl.pallas_call(
        functools.partial(_body, P1, P2, R1, R2),
        grid=(P1 + P2,),
        in_specs=[
            pl.BlockSpec((n_skills, d), lambda i: (0, 0)),
            pl.BlockSpec(memory_space=pltpu.MemorySpace.HBM),
            pl.BlockSpec(memory_space=pltpu.MemorySpace.HBM),
        ],
        out_specs=pl.BlockSpec((n_skills, d), lambda i: (0, 0)),
        out_shape=jax.ShapeDtypeStruct((n_skills, d), jnp.float32),
        scratch_shapes=[
            pltpu.VMEM((NBUF, R1, n_skills), jnp.float32),
            pltpu.VMEM((NBUF, R2, n_edges), jnp.float32),
            pltpu.VMEM((n_skills, d), jnp.bfloat16),
            pltpu.VMEM((n_edges, d), jnp.bfloat16),
            pltpu.SemaphoreType.DMA((NBUF,)),
            pltpu.SemaphoreType.DMA((NBUF,)),
        ],
    )(skill_embs, HG_qc, HG_cq)


# speedup vs baseline: 1.0163x; 1.0163x over previous
import functools

import jax
import jax.numpy as jnp
from jax.experimental import pallas as pl
from jax.experimental.pallas import tpu as pltpu

NBUF = 4
LOOKAHEAD = 3


def _body(P1, P2, R1, R2, e_ref, qc_ref, cq_ref, out_ref,
          pool1, pool2, msg_ref, sem1, sem2):
    i = pl.program_id(0)

    H1, H2 = R1 // 2, R2 // 2

    class _Pair:
        def __init__(self, a, b):
            self._cps = (a, b)

        def start(self):
            for c in self._cps:
                c.start()

        def wait(self):
            for c in self._cps:
                c.wait()

    def qc_copy(b):
        slot = jax.lax.rem(b, NBUF)
        return _Pair(
            pltpu.make_async_copy(
                qc_ref.at[pl.ds(b * R1, H1), :],
                pool1.at[slot, pl.ds(0, H1), :], sem1.at[slot, 0],
            ),
            pltpu.make_async_copy(
                qc_ref.at[pl.ds(b * R1 + H1, H1), :],
                pool1.at[slot, pl.ds(H1, H1), :], sem1.at[slot, 1],
            ),
        )

    def cq_copy(b):
        slot = jax.lax.rem(b, NBUF)
        return _Pair(
            pltpu.make_async_copy(
                cq_ref.at[pl.ds(b * R2, H2), :],
                pool2.at[slot, pl.ds(0, H2), :], sem2.at[slot, 0],
            ),
            pltpu.make_async_copy(
                cq_ref.at[pl.ds(b * R2 + H2, H2), :],
                pool2.at[slot, pl.ds(H2, H2), :], sem2.at[slot, 1],
            ),
        )

    @pl.when(i == 0)
    def _warmup():
        for b in range(min(LOOKAHEAD, P1)):
            qc_copy(b).start()

    @pl.when(i < P1)
    def _p1():
        @pl.when(i + LOOKAHEAD < P1)
        def _():
            qc_copy(i + LOOKAHEAD).start()

        @pl.when(i >= P1 - min(LOOKAHEAD, P2))
        def _():
            cq_copy(i - (P1 - min(LOOKAHEAD, P2))).start()

        qc_copy(i).wait()
        acc = jnp.dot(
            pool1[jax.lax.rem(i, NBUF)].astype(jnp.bfloat16),
            e_ref[...],
            preferred_element_type=jnp.float32,
        )
        msg_ref[pl.ds(i * R1, R1), :] = acc.astype(jnp.bfloat16)

    @pl.when(i >= P1)
    def _p2():
        j = i - P1

        @pl.when(j + LOOKAHEAD < P2)
        def _():
            cq_copy(j + LOOKAHEAD).start()

        cq_copy(j).wait()
        out_ref[pl.ds(j * R2, R2), :] = jnp.dot(
            pool2[jax.lax.rem(j, NBUF)].astype(jnp.bfloat16),
            msg_ref[...],
            preferred_element_type=jnp.float32,
        )


@jax.jit
def kernel(skill_embs, HG_qc, HG_cq):
    n_edges, n_skills = HG_qc.shape
    d = skill_embs.shape[1]
    R1 = 64
    R2 = 256
    P1 = n_edges // R1
    P2 = n_skills // R2

    return pl.pallas_call(
        functools.partial(_body, P1, P2, R1, R2),
        grid=(P1 + P2,),
        in_specs=[
            pl.BlockSpec((n_skills, d), lambda i: (0, 0)),
            pl.BlockSpec(memory_space=pltpu.MemorySpace.HBM),
            pl.BlockSpec(memory_space=pltpu.MemorySpace.HBM),
        ],
        out_specs=pl.BlockSpec((n_skills, d), lambda i: (0, 0)),
        out_shape=jax.ShapeDtypeStruct((n_skills, d), jnp.float32),
        scratch_shapes=[
            pltpu.VMEM((NBUF, R1, n_skills), jnp.float32),
            pltpu.VMEM((NBUF, R2, n_edges), jnp.float32),
            pltpu.VMEM((n_edges, d), jnp.bfloat16),
            pltpu.SemaphoreType.DMA((NBUF, 2)),
            pltpu.SemaphoreType.DMA((NBUF, 2)),
        ],
    )(skill_embs.astype(jnp.bfloat16), HG_qc, HG_cq)
